# baseline (speedup 1.0000x reference)
import functools

import jax
import jax.numpy as jnp
from jax import lax
from jax.experimental import pallas as pl
from jax.experimental.pallas import tpu as pltpu
from jax.experimental.pallas import tpu_sc as plsc

N0 = 10000
N1 = 20000
E = 640000
H = 128

RB0 = 2000
RB1 = 2000


def _mm_body(x_ref, w_ref, b_ref, o_ref):
    x = x_ref[...].astype(jnp.bfloat16)
    w = w_ref[...].astype(jnp.bfloat16)
    o_ref[...] = (jnp.dot(x, w, preferred_element_type=jnp.float32)
                  + b_ref[...])


def _proj_lin_body(x_ref, w0_ref, b0_ref, w1_ref, b1_ref, o_ref):
    x = x_ref[...].astype(jnp.bfloat16)
    w0 = w0_ref[...].astype(jnp.bfloat16)
    h = jnp.dot(x, w0, preferred_element_type=jnp.float32) + b0_ref[...]
    w1 = w1_ref[...].astype(jnp.bfloat16)
    o_ref[...] = (jnp.dot(h.astype(jnp.bfloat16), w1,
                          preferred_element_type=jnp.float32) + b1_ref[...])


def _proj_lin(x, W0, b0, W1, b1):
    n, f = x.shape
    return pl.pallas_call(
        _proj_lin_body,
        grid=(n // RB0,),
        in_specs=[
            pl.BlockSpec((RB0, f), lambda i: (i, 0)),
            pl.BlockSpec((f, H), lambda i: (0, 0)),
            pl.BlockSpec((1, H), lambda i: (0, 0)),
            pl.BlockSpec((H, H), lambda i: (0, 0)),
            pl.BlockSpec((1, H), lambda i: (0, 0)),
        ],
        out_specs=pl.BlockSpec((RB0, H), lambda i: (i, 0)),
        out_shape=jax.ShapeDtypeStruct((n, H), jnp.float32),
    )(x, W0, b0.reshape(1, H), W1, b1.reshape(1, H))


def _combine_mm_body(a_ref, m_ref, d_ref, w_ref, b_ref, o_ref):
    deg = d_ref[0, :, 0:1] + d_ref[1, :, 0:1]
    r = 1.0 / jnp.maximum(deg, 1.0)
    m = m_ref[0] + m_ref[1]
    x = jnp.maximum(a_ref[...] + m * r, 0.0)
    w = w_ref[...].astype(jnp.bfloat16)
    o_ref[...] = (jnp.dot(x.astype(jnp.bfloat16), w,
                          preferred_element_type=jnp.float32) + b_ref[...])


def _combine_mm(a, msgp, degp, W, b):
    return pl.pallas_call(
        _combine_mm_body,
        grid=(N0 // RB0,),
        in_specs=[
            pl.BlockSpec((RB0, H), lambda i: (i, 0)),
            pl.BlockSpec((2, RB0, H), lambda i: (0, i, 0)),
            pl.BlockSpec((2, RB0, H), lambda i: (0, i, 0)),
            pl.BlockSpec((H, H), lambda i: (0, 0)),
            pl.BlockSpec((1, H), lambda i: (0, 0)),
        ],
        out_specs=pl.BlockSpec((RB0, H), lambda i: (i, 0)),
        out_shape=jax.ShapeDtypeStruct((N0, H), jnp.float32),
    )(a, msgp, degp, W, b.reshape(1, H))


def _combine_sum_body(a_ref, m_ref, d_ref, s_ref):
    i = pl.program_id(0)
    deg = d_ref[0, :, 0:1] + d_ref[1, :, 0:1]
    r = 1.0 / jnp.maximum(deg, 1.0)
    x = a_ref[...] + (m_ref[0] + m_ref[1]) * r

    @pl.when(i == 0)
    def _():
        s_ref[...] = jnp.zeros_like(s_ref)
    s_ref[...] += jnp.sum(x, axis=0, keepdims=True)


def _combine_sum(a, msgp, degp):
    return pl.pallas_call(
        _combine_sum_body,
        grid=(N0 // RB0,),
        in_specs=[
            pl.BlockSpec((RB0, H), lambda i: (i, 0)),
            pl.BlockSpec((2, RB0, H), lambda i: (0, i, 0)),
            pl.BlockSpec((2, RB0, H), lambda i: (0, i, 0)),
        ],
        out_specs=pl.BlockSpec((1, H), lambda i: (0, 0)),
        out_shape=jax.ShapeDtypeStruct((1, H), jnp.float32),
    )(a, msgp, degp)


def _colsum_body(x_ref, s_ref):
    i = pl.program_id(0)

    @pl.when(i == 0)
    def _():
        s_ref[...] = jnp.zeros_like(s_ref)
    s_ref[...] += jnp.sum(x_ref[...], axis=0, keepdims=True)


def _colsum(x, rb):
    n = x.shape[0]
    return pl.pallas_call(
        _colsum_body,
        grid=(n // rb,),
        in_specs=[pl.BlockSpec((rb, H), lambda i: (i, 0))],
        out_specs=pl.BlockSpec((1, H), lambda i: (0, 0)),
        out_shape=jax.ShapeDtypeStruct((1, H), jnp.float32),
    )(x)


def _head_body(s0_ref, s1_ref, w0_ref, w1_ref, b_ref, o_ref):
    w0 = w0_ref[...].astype(jnp.bfloat16).astype(jnp.float32)
    w1 = w1_ref[...].astype(jnp.bfloat16).astype(jnp.float32)
    m0 = jnp.sum(s0_ref[...] * w0) / N0
    m1 = jnp.sum(s1_ref[...] * w1) / N1
    o_ref[...] = (m0 + m1).reshape(1, 1) + b_ref[...]


def _head(s0, s1, Wo0, Wo1, bo0, bo1):
    out = pl.pallas_call(
        _head_body,
        in_specs=[pl.BlockSpec((1, H), lambda: (0, 0))] * 4 +
                 [pl.BlockSpec((1, 1), lambda: (0, 0))],
        out_specs=pl.BlockSpec((1, 1), lambda: (0, 0)),
        out_shape=jax.ShapeDtypeStruct((1, 1), jnp.float32),
    )(s0, s1, Wo0.reshape(1, H), Wo1.reshape(1, H),
      (bo0 + bo1).reshape(1, 1))
    return out.reshape(1)


CW = 128
NCHUNK = E // CW
NTILES = 16
CPT = (NCHUNK + NTILES - 1) // NTILES
X1H = N1 // 2
MSGH = N0 // 2


DCH = NCHUNK // 2


def _deg_kernel():
    mesh = plsc.VectorSubcoreMesh(core_axis_name="c", subcore_axis_name="s")
    out_type = [jax.ShapeDtypeStruct((2, N0, H), jnp.float32)]
    scratch = [
        pltpu.VMEM_SHARED((N0 + 8, H), jnp.float32),
    ] + [pltpu.VMEM((1, CW), jnp.int32)] * 8 \
      + [pltpu.VMEM((CW, H), jnp.float32)] \
      + [pltpu.SemaphoreType.DMA] * 8

    @functools.partial(pl.kernel, out_type=out_type, mesh=mesh,
                       scratch_types=scratch)
    def k(vidx, z, ones_h, deg_out, sh,
          vb0, vb1, vb2, vb3, db0, db1, db2, db3, onesbuf,
          si0, si1, si2, si3, ss0, ss1, ss2, ss3):
        c = lax.axis_index("c")
        s = lax.axis_index("s")
        vb = (vb0, vb1, vb2, vb3)
        db = (db0, db1, db2, db3)
        si = (si0, si1, si2, si3)
        ss = (ss0, ss1, ss2, ss3)
        cpt = (DCH + NTILES - 1) // NTILES
        NIT = (cpt + 4 + 3) // 4

        @pl.when(s < 10)
        def _():
            pltpu.sync_copy(z, sh.at[pl.ds(s * 1000, 1000)])
        pltpu.sync_copy(ones_h, onesbuf)
        plsc.subcore_barrier()

        base = c * DCH

        def chunk(i):
            return base + s + i * NTILES

        def vld(i):
            return (s + i * NTILES) < DCH

        def issue_idx(i, q):
            pltpu.async_copy(vidx.at[chunk(i)], vb[q], si[q])

        for p in range(3):
            @pl.when(vld(p))
            def _():
                issue_idx(p, p)

        def body(i4, carry):
            for u in range(4):
                i = 4 * i4 + u
                qb = u
                q3 = (u + 3) % 4

                @pl.when((i >= 4) & vld(i - 4))
                def _():
                    pltpu.make_async_copy(
                        onesbuf, sh.at[pl.ds(0, CW)], ss[qb]).wait()

                @pl.when(vld(i))
                def _():
                    pltpu.make_async_copy(vidx.at[0], vb[qb], si[qb]).wait()
                    for t in range(CW // 16):
                        sl = pl.ds(t * 16, 16)
                        db[qb][0, sl] = vb[qb][0, sl]
                    pltpu.async_copy(onesbuf, sh.at[db[qb].at[0]], ss[qb],
                                     add=True)

                @pl.when(vld(i + 3))
                def _():
                    issue_idx(i + 3, q3)
            return carry

        lax.fori_loop(0, NIT, body, 0)
        plsc.subcore_barrier()

        @pl.when(s < 10)
        def _():
            pltpu.sync_copy(sh.at[pl.ds(s * 1000, 1000)],
                            deg_out.at[c].at[pl.ds(s * 1000, 1000)])

    return k


def _sc_layer_kernel():
    mesh = plsc.VectorSubcoreMesh(core_axis_name="c", subcore_axis_name="s")
    out_type = [
        jax.ShapeDtypeStruct((N1, H), jnp.float32),
        jax.ShapeDtypeStruct((2, N0, H), jnp.float32),
    ]
    scratch = [
        pltpu.VMEM_SHARED((X1H + 8, H), jnp.float32),
    ] + [pltpu.VMEM((2, CW), jnp.int32)] * 4 \
      + [pltpu.VMEM((1, CW), jnp.int32)] * 4 \
      + [pltpu.VMEM((CW, H), jnp.float32)] * 2 \
      + [pltpu.SemaphoreType.DMA] * 8

    @functools.partial(pl.kernel, out_type=out_type, mesh=mesh,
                       scratch_types=scratch)
    def k(a0, midx, z, x1_out, msg_out,
          sh, ib0, ib1, ib2, ib3, lb0, lb1, lb2, lb3, r0, r1,
          si0, si1, si2, si3, sg0, sg1, ss0, ss1):
        c = lax.axis_index("c")
        s = lax.axis_index("s")
        ib, lb = (ib0, ib1, ib2, ib3), (lb0, lb1, lb2, lb3)
        rws = (r0, r1)
        si = (si0, si1, si2, si3)
        sg, ss = (sg0, sg1), (ss0, ss1)

        def run_pass(table, gsel, lo, size, base, count):
            cpt = (count + NTILES - 1) // NTILES
            nit = (cpt + 2 + 3) // 4

            def chunk(i):
                return base + s + i * NTILES

            def vld(i):
                return (s + i * NTILES) < count

            def issue_idx(i, q):
                pltpu.async_copy(midx.at[chunk(i)], ib[q], si[q])

            grow, srow = (0, 1) if gsel == 0 else (1, 0)
            for p in range(3):
                @pl.when(vld(p))
                def _():
                    issue_idx(p, p)

            def body(i4, carry):
                for u in range(4):
                    i = 4 * i4 + u
                    qb = u
                    q3 = (u + 3) % 4
                    rb = u % 2
                    ro = 1 - rb

                    @pl.when((i >= 2) & vld(i - 2))
                    def _():
                        pltpu.make_async_copy(
                            rws[rb], sh.at[pl.ds(0, CW)], ss[rb]).wait()

                    @pl.when(vld(i))
                    def _():
                        pltpu.make_async_copy(midx.at[0], ib[qb],
                                              si[qb]).wait()
                        for t in range(CW // 16):
                            sl = pl.ds(t * 16, 16)
                            lx = ib[qb][srow, sl] - lo
                            ok = (lx >= 0) & (lx < size)
                            lb[qb][0, sl] = jnp.where(ok, lx, size + (t % 8))
                        pltpu.async_copy(table.at[ib[qb].at[grow]], rws[rb],
                                         sg[rb])

                    @pl.when((i >= 1) & vld(i - 1))
                    def _():
                        pltpu.make_async_copy(
                            table.at[pl.ds(0, CW)], rws[ro], sg[ro]).wait()
                        pltpu.async_copy(rws[ro], sh.at[lb[q3].at[0]],
                                         ss[ro], add=True)

                    @pl.when(vld(i + 3))
                    def _():
                        issue_idx(i + 3, q3)
                return carry

            lax.fori_loop(0, nit, body, 0)

        @pl.when(s < 10)
        def _():
            pltpu.sync_copy(z, sh.at[pl.ds(s * 1000, 1000)])
        plsc.subcore_barrier()

        run_pass(a0, 0, c * X1H, X1H, 0, NCHUNK)
        plsc.subcore_barrier()

        @pl.when(s < 10)
        def _():
            pltpu.sync_copy(sh.at[pl.ds(s * 1000, 1000)],
                            x1_out.at[pl.ds(c * X1H + s * 1000, 1000)])

        @pl.when(s < 10)
        def _():
            pltpu.sync_copy(z, sh.at[pl.ds(s * 1000, 1000)])
        plsc.subcore_barrier()

        run_pass(x1_out, 1, 0, N0, c * DCH, DCH)
        plsc.subcore_barrier()

        @pl.when(s < 10)
        def _():
            pltpu.sync_copy(sh.at[pl.ds(s * 1000, 1000)],
                            msg_out.at[c].at[pl.ds(s * 1000, 1000)])

    return k


_sc_deg = _deg_kernel()
_sc_layer_k = _sc_layer_kernel()


def _sc_layer(a0, m_idx3, z):
    return _sc_layer_k(a0, m_idx3, z)


def kernel(x_0, x_1, vertex_idx, hyperedge_idx,
           W0_in, b0_in, W1_in, b1_in, Wl0, bl0, Wl1, bl1,
           Wo0, bo0, Wo1, bo1):
    v_idx3 = vertex_idx.astype(jnp.int32).reshape(NCHUNK, 1, CW)
    e_idx3 = hyperedge_idx.astype(jnp.int32).reshape(NCHUNK, 1, CW)
    m_idx3 = jnp.concatenate([v_idx3, e_idx3], axis=1)
    z = jnp.zeros((1000, H), jnp.float32)
    ones_h = jnp.ones((CW, H), jnp.float32)
    degp = _sc_deg(v_idx3, z, ones_h)[0]

    a1 = _proj_lin(x_0, W0_in, b0_in, Wl0, bl0)
    _, msg1 = _sc_layer(a1, m_idx3, z)
    a2 = _combine_mm(a1, msg1, degp, Wl1, bl1)
    x1_2, msg2 = _sc_layer(a2, m_idx3, z)
    s0 = _combine_sum(a2, msg2, degp)

    s1 = _colsum(x1_2, RB1)
    return _head(s0, s1, Wo0, Wo1, bo0, bo1)
